# baseline (speedup 1.0000x reference)
import jax
import jax.numpy as jnp
from jax.experimental import pallas as pl
from jax.experimental.pallas import tpu as pltpu

IN_DIM = 1024
OUT_DIM = 1024
K = 2048
B_TOK = 8192
TM = 512


def _prep_body(A_ref, B_ref, An_ref, Bb_ref):
    a = A_ref[...]
    inv = jax.lax.rsqrt(jnp.sum(a * a, axis=0, keepdims=True))
    An_ref[...] = (a * inv).astype(jnp.bfloat16)
    Bb_ref[...] = B_ref[...].astype(jnp.bfloat16)


def _fused_body(x_ref, An_ref, Bb_ref, out_ref, inner_ref):
    inner = jnp.dot(x_ref[...].astype(jnp.bfloat16), An_ref[...],
                    preferred_element_type=jnp.float32)
    inner_ref[...] = inner
    out_ref[...] = jnp.dot(inner.astype(jnp.bfloat16), Bb_ref[...],
                           preferred_element_type=jnp.float32)


def kernel(x, A, Bm):
    An, Bb = pl.pallas_call(
        _prep_body,
        in_specs=[
            pl.BlockSpec((IN_DIM, K), lambda: (0, 0)),
            pl.BlockSpec((K, OUT_DIM), lambda: (0, 0)),
        ],
        out_specs=[
            pl.BlockSpec((IN_DIM, K), lambda: (0, 0)),
            pl.BlockSpec((K, OUT_DIM), lambda: (0, 0)),
        ],
        out_shape=[
            jax.ShapeDtypeStruct((IN_DIM, K), jnp.bfloat16),
            jax.ShapeDtypeStruct((K, OUT_DIM), jnp.bfloat16),
        ],
    )(A, Bm)

    n_tiles = B_TOK // TM
    out, inner = pl.pallas_call(
        _fused_body,
        grid=(n_tiles,),
        in_specs=[
            pl.BlockSpec((TM, IN_DIM), lambda i: (i, 0)),
            pl.BlockSpec((IN_DIM, K), lambda i: (0, 0)),
            pl.BlockSpec((K, OUT_DIM), lambda i: (0, 0)),
        ],
        out_specs=[
            pl.BlockSpec((TM, OUT_DIM), lambda i: (i, 0)),
            pl.BlockSpec((TM, K), lambda i: (i, 0)),
        ],
        out_shape=[
            jax.ShapeDtypeStruct((B_TOK, OUT_DIM), jnp.float32),
            jax.ShapeDtypeStruct((B_TOK, K), jnp.float32),
        ],
        compiler_params=pltpu.CompilerParams(
            dimension_semantics=("parallel",),
        ),
    )(x, An, Bb)
    return (out, inner)


# speedup vs baseline: 1.0081x; 1.0081x over previous
import jax
import jax.numpy as jnp
from jax.experimental import pallas as pl
from jax.experimental.pallas import tpu as pltpu

IN_DIM = 1024
OUT_DIM = 1024
K = 2048
B_TOK = 8192
TM = 1024


def _prep_body(A_ref, B_ref, An_ref, Bb_ref):
    a = A_ref[...]
    inv = jax.lax.rsqrt(jnp.sum(a * a, axis=0, keepdims=True))
    An_ref[...] = (a * inv).astype(jnp.bfloat16)
    Bb_ref[...] = B_ref[...].astype(jnp.bfloat16)


def _fused_body(x_ref, An_ref, Bb_ref, out_ref, inner_ref):
    inner = jnp.dot(x_ref[...].astype(jnp.bfloat16), An_ref[...],
                    preferred_element_type=jnp.float32)
    inner_ref[...] = inner
    out_ref[...] = jnp.dot(inner.astype(jnp.bfloat16), Bb_ref[...],
                           preferred_element_type=jnp.float32)


def kernel(x, A, Bm):
    An, Bb = pl.pallas_call(
        _prep_body,
        in_specs=[
            pl.BlockSpec((IN_DIM, K), lambda: (0, 0)),
            pl.BlockSpec((K, OUT_DIM), lambda: (0, 0)),
        ],
        out_specs=[
            pl.BlockSpec((IN_DIM, K), lambda: (0, 0)),
            pl.BlockSpec((K, OUT_DIM), lambda: (0, 0)),
        ],
        out_shape=[
            jax.ShapeDtypeStruct((IN_DIM, K), jnp.bfloat16),
            jax.ShapeDtypeStruct((K, OUT_DIM), jnp.bfloat16),
        ],
    )(A, Bm)

    n_tiles = B_TOK // TM
    out, inner = pl.pallas_call(
        _fused_body,
        grid=(n_tiles,),
        in_specs=[
            pl.BlockSpec((TM, IN_DIM), lambda i: (i, 0)),
            pl.BlockSpec((IN_DIM, K), lambda i: (0, 0)),
            pl.BlockSpec((K, OUT_DIM), lambda i: (0, 0)),
        ],
        out_specs=[
            pl.BlockSpec((TM, OUT_DIM), lambda i: (i, 0)),
            pl.BlockSpec((TM, K), lambda i: (i, 0)),
        ],
        out_shape=[
            jax.ShapeDtypeStruct((B_TOK, OUT_DIM), jnp.float32),
            jax.ShapeDtypeStruct((B_TOK, K), jnp.float32),
        ],
        compiler_params=pltpu.CompilerParams(
            dimension_semantics=("parallel",),
        ),
    )(x, An, Bb)
    return (out, inner)
